# baseline (speedup 1.0000x reference)
import jax
import jax.numpy as jnp
from jax import lax
from jax.experimental import pallas as pl
from jax.experimental.pallas import tpu as pltpu
from jax.experimental.pallas import tpu_sc as plsc

N = 10000
D = 128
E = 320000
NC = 2
NS = 16
NW = NC * NS
L = 16
EPW = E // NW
K = 112
NBLK = EPW // K
REM = EPW - NBLK * K
RB = 80
NRB = N // RB
MROW = 1000


def _matmul_body(x_ref, w_ref, q1_ref, q2_ref, h_ref, a1_ref, a2_ref):
    x = x_ref[...]
    h = jnp.dot(x, w_ref[...], preferred_element_type=jnp.float32)
    h_ref[...] = h
    a1_ref[...] = jnp.sum(h * q1_ref[...], axis=1, keepdims=True)
    a2_ref[...] = jnp.sum(h * q2_ref[...], axis=1, keepdims=True)


NSLOT = 3
NREF = 6


def _edge_body(h_hbm, a1_hbm, a2_hbm, src_hbm, dst_hbm,
               acc_hbm, den_hbm, *rest):
    c = lax.axis_index("c")
    s = lax.axis_index("s")
    wid = c * NS + s

    slot_refs = [rest[NREF * b:NREF * b + NREF] for b in range(NSLOT)]
    acc_sp = rest[NREF * NSLOT]
    den_sp = rest[NREF * NSLOT + 1]
    a1_sp = rest[NREF * NSLOT + 2]
    a2_sp = rest[NREF * NSLOT + 3]
    sem_base = NREF * NSLOT + 4
    slot_sems = [rest[sem_base + 5 * b:sem_base + 5 * b + 5]
                 for b in range(NSLOT)]
    src_r, dst_r, w_r, a1r, a2r = rest[sem_base + 5 * NSLOT:]
    bufs = tuple(tuple(slot_refs[b]) + tuple(slot_sems[b]) for b in range(NSLOT))
    src0, dst0, w0, rows0 = slot_refs[0][:4]
    w1 = slot_refs[1][2]
    w2 = slot_refs[2][2]

    zeros = jnp.zeros((L,), jnp.float32)

    def _zrow(r, carry):
        for cc in range(D // L):
            rows0[r, pl.ds(cc * L, L)] = zeros
        return carry

    lax.fori_loop(0, K, _zrow, 0)
    for g in range(K // L):
        w0[pl.ds(g * L, L)] = zeros

    ZB = 624
    _offs = (0, 112, 224, 336, 448, 528)
    zsem = bufs[0][NREF + 1]
    zds = [pltpu.async_copy(rows0, acc_sp.at[pl.ds(s * ZB + o, K)], zsem)
           for o in _offs]
    for j2 in range(pl.cdiv(NRB, NS)):
        j = s + j2 * NS

        @pl.when(j < NRB)
        def _():
            pltpu.sync_copy(w0.at[pl.ds(0, RB)], den_sp.at[pl.ds(j * RB, RB)])
            pltpu.sync_copy(a1_hbm.at[pl.ds(j * RB, RB)], w1.at[pl.ds(0, RB)])
            pltpu.sync_copy(w1.at[pl.ds(0, RB)], a1_sp.at[pl.ds(j * RB, RB)])
            pltpu.sync_copy(a2_hbm.at[pl.ds(j * RB, RB)], w2.at[pl.ds(0, RB)])
            pltpu.sync_copy(w2.at[pl.ds(0, RB)], a2_sp.at[pl.ds(j * RB, RB)])

    for d in zds:
        d.wait()

    plsc.subcore_barrier()

    base = wid * EPW

    def _load_idx(i, b):
        src_v, dst_v, sem = bufs[b][0], bufs[b][1], bufs[b][NREF]
        off = pl.multiple_of(base + i * K, 8)
        d1 = pltpu.async_copy(src_hbm.at[pl.ds(off, K)], src_v, sem)
        d2 = pltpu.async_copy(dst_hbm.at[pl.ds(off, K)], dst_v, sem)
        return d1, d2

    def _start_gather(b):
        src_v, dst_v, rows_v = bufs[b][0], bufs[b][1], bufs[b][3]
        a1b, a2b = bufs[b][4], bufs[b][5]
        g = pltpu.async_copy(h_hbm.at[src_v], rows_v, bufs[b][NREF + 1])
        ga1 = pltpu.async_copy(a1_sp.at[src_v], a1b, bufs[b][NREF + 4])
        ga2 = pltpu.async_copy(a2_sp.at[dst_v], a2b, bufs[b][NREF + 4])
        return (g, ga1, ga2)

    def _compute_w(b):
        w_v, a1b, a2b = bufs[b][2], bufs[b][4], bufs[b][5]
        for g in range(K // L):
            sl = pl.ds(g * L, L)
            logit = a1b[sl] + a2b[sl]
            logit = jnp.where(logit >= 0.0, logit, 0.2 * logit)
            w_v[sl] = jnp.exp(logit)

    def _scale(b):
        w_v, rows_v = bufs[b][2], bufs[b][3]

        def _body(g2, carry2):
            w16 = w_v[pl.ds(g2 * L, L)]
            for r2 in range(L):
                r = g2 * L + r2
                wr = w16[r2]
                for cc in range(D // L):
                    sl = pl.ds(cc * L, L)
                    rows_v[r, sl] = rows_v[r, sl] * wr
            return carry2

        lax.fori_loop(0, K // L, _body, 0)

    def _start_scatter(b):
        dst_v, w_v, rows_v = bufs[b][1], bufs[b][2], bufs[b][3]
        pltpu.async_copy(rows_v, acc_sp.at[dst_v], bufs[b][NREF + 2], add=True)
        pltpu.async_copy(w_v, den_sp.at[dst_v], bufs[b][NREF + 3], add=True)

    def _wait_scatter(b):
        dst_v, w_v, rows_v = bufs[b][1], bufs[b][2], bufs[b][3]
        pltpu.make_async_copy(rows_v, acc_sp.at[dst_v], bufs[b][NREF + 2]).wait()
        pltpu.make_async_copy(w_v, den_sp.at[dst_v], bufs[b][NREF + 3]).wait()

    def _front(i, b):
        d1, d2 = _load_idx(i, b)
        d1.wait()
        d2.wait()
        return _start_gather(b)

    def _back(b, g):
        g[1].wait()
        g[2].wait()
        _compute_w(b)
        g[0].wait()
        _scale(b)
        _start_scatter(b)

    NQ = NBLK // NSLOT
    gs = [_front(j, j) for j in range(NSLOT)]
    for j in range(NSLOT):
        _back(j, gs[j])

    def _quad(q, carry):
        gs2 = []
        for j in range(NSLOT):
            _wait_scatter(j)
            gs2.append(_front(q * NSLOT + j, j))
        for j in range(NSLOT):
            _back(j, gs2[j])
        return carry

    lax.fori_loop(1, NQ, _quad, 0)

    for i in range(NSLOT * NQ, NBLK):
        b = i - NSLOT * NQ
        _wait_scatter(b)
        g = _front(i, b)
        _back(b, g)

    for b in range(NSLOT):
        _wait_scatter(b)

    rows_t = slot_refs[0][3]
    sem_i, sem_g, sem_r2, sem_w2, sem_a = slot_sems[0]
    off_r = pl.multiple_of(base + NBLK * K, 8)
    d1 = pltpu.async_copy(src_hbm.at[pl.ds(off_r, REM)], src_r, sem_i)
    d2 = pltpu.async_copy(dst_hbm.at[pl.ds(off_r, REM)], dst_r, sem_i)
    d1.wait()
    d2.wait()
    g_r = pltpu.async_copy(h_hbm.at[src_r], rows_t.at[pl.ds(0, REM)], sem_g)
    ga1 = pltpu.async_copy(a1_sp.at[src_r], a1r, sem_a)
    ga2 = pltpu.async_copy(a2_sp.at[dst_r], a2r, sem_a)
    ga1.wait()
    ga2.wait()
    for g in range(REM // L):
        sl = pl.ds(g * L, L)
        logit = a1r[sl] + a2r[sl]
        logit = jnp.where(logit >= 0.0, logit, 0.2 * logit)
        w_r[sl] = jnp.exp(logit)
    g_r.wait()
    for g2 in range(REM // L):
        w16 = w_r[pl.ds(g2 * L, L)]
        for r2 in range(L):
            r = g2 * L + r2
            wr = w16[r2]
            for cc in range(D // L):
                sl = pl.ds(cc * L, L)
                rows_t[r, sl] = rows_t[r, sl] * wr
    sr = pltpu.async_copy(rows_t.at[pl.ds(0, REM)], acc_sp.at[dst_r], sem_r2,
                          add=True)
    sw = pltpu.async_copy(w_r, den_sp.at[dst_r], sem_w2, add=True)
    sr.wait()
    sw.wait()

    plsc.subcore_barrier()

    rowsbufs = [slot_refs[b][3] for b in range(NSLOT)]
    rsems = [bufs[b][NREF + 1] for b in range(NSLOT)]
    wsems = [bufs[b][NREF + 2] for b in range(NSLOT)]

    def _rd_acc(k):
        return pltpu.async_copy(acc_sp.at[pl.ds(s * ZB + _offs[k], K)],
                                rowsbufs[k % NSLOT], rsems[k % NSLOT])

    rds = {k: _rd_acc(k) for k in range(NSLOT)}
    for k in range(len(_offs)):
        rds[k].wait()
        wrk = pltpu.async_copy(rowsbufs[k % NSLOT],
                               acc_hbm.at[c, pl.ds(s * ZB + _offs[k], K)],
                               wsems[k % NSLOT])
        if k + NSLOT < len(_offs):
            wrk.wait()
            rds[k + NSLOT] = _rd_acc(k + NSLOT)
        else:
            wrk.wait()

    for j2 in range(pl.cdiv(NRB, NS)):
        j = s + j2 * NS

        @pl.when(j < NRB)
        def _():
            pltpu.sync_copy(den_sp.at[pl.ds(j * RB, RB)], w0.at[pl.ds(0, RB)])
            pltpu.sync_copy(w0.at[pl.ds(0, RB)],
                            den_hbm.at[pl.ds(c * N + j * RB, RB)])


def _fin_body(acc_ref, den_ref, out_ref):
    a = acc_ref[0] + acc_ref[1]
    rd = 1.0 / (den_ref[0] + den_ref[1] + 1e-10)
    out_ref[...] = jnp.maximum(a * rd, 0.0)


def kernel(node_feat_protein, node_feat_ligand, edge_index, W, query):
    x = jnp.concatenate([node_feat_protein, node_feat_ligand], axis=0)
    q1 = query[:D].reshape(1, D)
    q2 = query[D:].reshape(1, D)

    h, a1, a2 = pl.pallas_call(
        _matmul_body,
        grid=(N // MROW,),
        in_specs=[
            pl.BlockSpec((MROW, D), lambda i: (i, 0)),
            pl.BlockSpec((D, D), lambda i: (0, 0)),
            pl.BlockSpec((1, D), lambda i: (0, 0)),
            pl.BlockSpec((1, D), lambda i: (0, 0)),
        ],
        out_specs=[
            pl.BlockSpec((MROW, D), lambda i: (i, 0)),
            pl.BlockSpec((MROW, 1), lambda i: (i, 0)),
            pl.BlockSpec((MROW, 1), lambda i: (i, 0)),
        ],
        out_shape=[
            jax.ShapeDtypeStruct((N, D), jnp.float32),
            jax.ShapeDtypeStruct((N, 1), jnp.float32),
            jax.ShapeDtypeStruct((N, 1), jnp.float32),
        ],
    )(x, W, q1, q2)

    a1 = a1.reshape(N)
    a2 = a2.reshape(N)
    src = edge_index[0].astype(jnp.int32)
    dst = edge_index[1].astype(jnp.int32)

    mesh = plsc.VectorSubcoreMesh(
        core_axis_name="c", subcore_axis_name="s", num_cores=NC, num_subcores=NS
    )

    sc_params = pltpu.CompilerParams(needs_layout_passes=False)

    edge_k = pl.kernel(
        _edge_body,
        compiler_params=sc_params,
        out_type=[
            jax.ShapeDtypeStruct((NC, N, D), jnp.float32),
            jax.ShapeDtypeStruct((NC * N,), jnp.float32),
        ],
        mesh=mesh,
        scratch_types=[
            pltpu.VMEM((K,), jnp.int32),
            pltpu.VMEM((K,), jnp.int32),
            pltpu.VMEM((K,), jnp.float32),
            pltpu.VMEM((K, D), jnp.float32),
            pltpu.VMEM((K,), jnp.float32),
            pltpu.VMEM((K,), jnp.float32),
        ] * NSLOT + [
            pltpu.VMEM_SHARED((N, D), jnp.float32),
            pltpu.VMEM_SHARED((N,), jnp.float32),
            pltpu.VMEM_SHARED((N,), jnp.float32),
            pltpu.VMEM_SHARED((N,), jnp.float32),
        ] + [pltpu.SemaphoreType.DMA] * (5 * NSLOT) + [
            pltpu.VMEM((REM,), jnp.int32),
            pltpu.VMEM((REM,), jnp.int32),
            pltpu.VMEM((REM,), jnp.float32),
            pltpu.VMEM((REM,), jnp.float32),
            pltpu.VMEM((REM,), jnp.float32),
        ],
    )
    acc, den = edge_k(h, a1, a2, src, dst)

    den3 = den.reshape(NC, N, 1)
    FR = 2000
    out = pl.pallas_call(
        _fin_body,
        grid=(N // FR,),
        in_specs=[
            pl.BlockSpec((NC, FR, D), lambda i: (0, i, 0)),
            pl.BlockSpec((NC, FR, 1), lambda i: (0, i, 0)),
        ],
        out_specs=pl.BlockSpec((FR, D), lambda i: (i, 0)),
        out_shape=jax.ShapeDtypeStruct((N, D), jnp.float32),
    )(acc, den3)
    return (out[:5000], out[5000:])


# speedup vs baseline: 1.0087x; 1.0087x over previous
import jax
import jax.numpy as jnp
from jax import lax
from jax.experimental import pallas as pl
from jax.experimental.pallas import tpu as pltpu
from jax.experimental.pallas import tpu_sc as plsc

N = 10000
D = 128
E = 320000
NC = 2
NS = 16
NW = NC * NS
L = 16
EPW = E // NW
K = 80
NBLK = EPW // K
RB = 80
NRB = N // RB
MROW = 1000


def _matmul_body(x_ref, w_ref, q1_ref, q2_ref, h_ref, a1_ref, a2_ref):
    x = x_ref[...]
    h = jnp.dot(x, w_ref[...], preferred_element_type=jnp.float32)
    h_ref[...] = h
    a1_ref[...] = jnp.sum(h * q1_ref[...], axis=1, keepdims=True)
    a2_ref[...] = jnp.sum(h * q2_ref[...], axis=1, keepdims=True)


NSLOT = 4
NREF = 6


def _edge_body(h_hbm, a1_hbm, a2_hbm, src_hbm, dst_hbm,
               acc_hbm, den_hbm, *rest):
    c = lax.axis_index("c")
    s = lax.axis_index("s")
    wid = c * NS + s

    slot_refs = [rest[NREF * b:NREF * b + NREF] for b in range(NSLOT)]
    acc_sp = rest[NREF * NSLOT]
    den_sp = rest[NREF * NSLOT + 1]
    a1_sp = rest[NREF * NSLOT + 2]
    a2_sp = rest[NREF * NSLOT + 3]
    sem_base = NREF * NSLOT + 4
    slot_sems = [rest[sem_base + 5 * b:sem_base + 5 * b + 5]
                 for b in range(NSLOT)]
    bufs = tuple(tuple(slot_refs[b]) + tuple(slot_sems[b]) for b in range(NSLOT))
    src0, dst0, w0, rows0 = slot_refs[0][:4]
    w1 = slot_refs[1][2]
    w2 = slot_refs[2][2]

    zeros = jnp.zeros((L,), jnp.float32)

    def _zrow(r, carry):
        for cc in range(D // L):
            rows0[r, pl.ds(cc * L, L)] = zeros
        return carry

    lax.fori_loop(0, K, _zrow, 0)
    for g in range(K // L):
        w0[pl.ds(g * L, L)] = zeros

    ZB = 624
    _offs = tuple(range(0, 8 * K, K))
    zsem = bufs[0][NREF + 1]
    zds = [pltpu.async_copy(rows0, acc_sp.at[pl.ds(s * ZB + o, K)], zsem)
           for o in _offs]
    for j2 in range(pl.cdiv(NRB, NS)):
        j = s + j2 * NS

        @pl.when(j < NRB)
        def _():
            pltpu.sync_copy(w0, den_sp.at[pl.ds(j * RB, RB)])
            pltpu.sync_copy(a1_hbm.at[pl.ds(j * RB, RB)], w1)
            pltpu.sync_copy(w1, a1_sp.at[pl.ds(j * RB, RB)])
            pltpu.sync_copy(a2_hbm.at[pl.ds(j * RB, RB)], w2)
            pltpu.sync_copy(w2, a2_sp.at[pl.ds(j * RB, RB)])

    for d in zds:
        d.wait()

    plsc.subcore_barrier()

    base = wid * EPW

    def _load_idx(i, b):
        src_v, dst_v, sem = bufs[b][0], bufs[b][1], bufs[b][NREF]
        off = pl.multiple_of(base + i * K, 8)
        d1 = pltpu.async_copy(src_hbm.at[pl.ds(off, K)], src_v, sem)
        d2 = pltpu.async_copy(dst_hbm.at[pl.ds(off, K)], dst_v, sem)
        return d1, d2

    def _start_gather(b):
        src_v, dst_v, rows_v = bufs[b][0], bufs[b][1], bufs[b][3]
        a1b, a2b = bufs[b][4], bufs[b][5]
        g = pltpu.async_copy(h_hbm.at[src_v], rows_v, bufs[b][NREF + 1])
        ga1 = pltpu.async_copy(a1_sp.at[src_v], a1b, bufs[b][NREF + 4])
        ga2 = pltpu.async_copy(a2_sp.at[dst_v], a2b, bufs[b][NREF + 4])
        return (g, ga1, ga2)

    def _compute_w(b):
        w_v, a1b, a2b = bufs[b][2], bufs[b][4], bufs[b][5]
        for g in range(K // L):
            sl = pl.ds(g * L, L)
            logit = a1b[sl] + a2b[sl]
            logit = jnp.where(logit >= 0.0, logit, 0.2 * logit)
            w_v[sl] = jnp.exp(logit)

    def _scale(b):
        w_v, rows_v = bufs[b][2], bufs[b][3]

        def _body(g2, carry2):
            w16 = w_v[pl.ds(g2 * L, L)]
            for r2 in range(L):
                r = g2 * L + r2
                wr = w16[r2]
                for cc in range(D // L):
                    sl = pl.ds(cc * L, L)
                    rows_v[r, sl] = rows_v[r, sl] * wr
            return carry2

        lax.fori_loop(0, K // L, _body, 0)

    def _start_scatter(b):
        dst_v, w_v, rows_v = bufs[b][1], bufs[b][2], bufs[b][3]
        pltpu.async_copy(rows_v, acc_sp.at[dst_v], bufs[b][NREF + 2], add=True)
        pltpu.async_copy(w_v, den_sp.at[dst_v], bufs[b][NREF + 3], add=True)

    def _wait_scatter(b):
        dst_v, w_v, rows_v = bufs[b][1], bufs[b][2], bufs[b][3]
        pltpu.make_async_copy(rows_v, acc_sp.at[dst_v], bufs[b][NREF + 2]).wait()
        pltpu.make_async_copy(w_v, den_sp.at[dst_v], bufs[b][NREF + 3]).wait()

    def _front(i, b):
        d1, d2 = _load_idx(i, b)
        d1.wait()
        d2.wait()
        return _start_gather(b)

    def _back(b, g):
        g[1].wait()
        g[2].wait()
        _compute_w(b)
        g[0].wait()
        _scale(b)
        _start_scatter(b)

    NQ = NBLK // NSLOT
    gs = [_front(j, j) for j in range(NSLOT)]
    for j in range(NSLOT):
        _back(j, gs[j])

    def _quad(q, carry):
        gs2 = []
        for j in range(NSLOT):
            _wait_scatter(j)
            gs2.append(_front(q * NSLOT + j, j))
        for j in range(NSLOT):
            _back(j, gs2[j])
        return carry

    lax.fori_loop(1, NQ, _quad, 0)

    for i in range(NSLOT * NQ, NBLK):
        b = i - NSLOT * NQ
        _wait_scatter(b)
        g = _front(i, b)
        _back(b, g)

    for b in range(NSLOT):
        _wait_scatter(b)

    plsc.subcore_barrier()

    rowsbufs = [slot_refs[b][3] for b in range(NSLOT)]
    rsems = [bufs[b][NREF + 1] for b in range(NSLOT)]
    wsems = [bufs[b][NREF + 2] for b in range(NSLOT)]

    def _rd_acc(k):
        return pltpu.async_copy(acc_sp.at[pl.ds(s * ZB + _offs[k], K)],
                                rowsbufs[k % NSLOT], rsems[k % NSLOT])

    rds = {k: _rd_acc(k) for k in range(NSLOT)}
    for k in range(len(_offs)):
        rds[k].wait()
        wrk = pltpu.async_copy(rowsbufs[k % NSLOT],
                               acc_hbm.at[c, pl.ds(s * ZB + _offs[k], K)],
                               wsems[k % NSLOT])
        if k + NSLOT < len(_offs):
            wrk.wait()
            rds[k + NSLOT] = _rd_acc(k + NSLOT)
        else:
            wrk.wait()

    for j2 in range(pl.cdiv(NRB, NS)):
        j = s + j2 * NS

        @pl.when(j < NRB)
        def _():
            pltpu.sync_copy(den_sp.at[pl.ds(j * RB, RB)], w0)
            pltpu.sync_copy(w0, den_hbm.at[pl.ds(c * N + j * RB, RB)])


def _fin_body(acc_ref, den_ref, out_ref):
    a = acc_ref[0] + acc_ref[1]
    rd = 1.0 / (den_ref[0] + den_ref[1] + 1e-10)
    out_ref[...] = jnp.maximum(a * rd, 0.0)


def kernel(node_feat_protein, node_feat_ligand, edge_index, W, query):
    x = jnp.concatenate([node_feat_protein, node_feat_ligand], axis=0)
    q1 = query[:D].reshape(1, D)
    q2 = query[D:].reshape(1, D)

    h, a1, a2 = pl.pallas_call(
        _matmul_body,
        grid=(N // MROW,),
        in_specs=[
            pl.BlockSpec((MROW, D), lambda i: (i, 0)),
            pl.BlockSpec((D, D), lambda i: (0, 0)),
            pl.BlockSpec((1, D), lambda i: (0, 0)),
            pl.BlockSpec((1, D), lambda i: (0, 0)),
        ],
        out_specs=[
            pl.BlockSpec((MROW, D), lambda i: (i, 0)),
            pl.BlockSpec((MROW, 1), lambda i: (i, 0)),
            pl.BlockSpec((MROW, 1), lambda i: (i, 0)),
        ],
        out_shape=[
            jax.ShapeDtypeStruct((N, D), jnp.float32),
            jax.ShapeDtypeStruct((N, 1), jnp.float32),
            jax.ShapeDtypeStruct((N, 1), jnp.float32),
        ],
    )(x, W, q1, q2)

    a1 = a1.reshape(N)
    a2 = a2.reshape(N)
    src = edge_index[0].astype(jnp.int32)
    dst = edge_index[1].astype(jnp.int32)

    mesh = plsc.VectorSubcoreMesh(
        core_axis_name="c", subcore_axis_name="s", num_cores=NC, num_subcores=NS
    )

    sc_params = pltpu.CompilerParams(needs_layout_passes=False)

    edge_k = pl.kernel(
        _edge_body,
        compiler_params=sc_params,
        out_type=[
            jax.ShapeDtypeStruct((NC, N, D), jnp.float32),
            jax.ShapeDtypeStruct((NC * N,), jnp.float32),
        ],
        mesh=mesh,
        scratch_types=[
            pltpu.VMEM((K,), jnp.int32),
            pltpu.VMEM((K,), jnp.int32),
            pltpu.VMEM((K,), jnp.float32),
            pltpu.VMEM((K, D), jnp.float32),
            pltpu.VMEM((K,), jnp.float32),
            pltpu.VMEM((K,), jnp.float32),
        ] * NSLOT + [
            pltpu.VMEM_SHARED((N, D), jnp.float32),
            pltpu.VMEM_SHARED((N,), jnp.float32),
            pltpu.VMEM_SHARED((N,), jnp.float32),
            pltpu.VMEM_SHARED((N,), jnp.float32),
        ] + [pltpu.SemaphoreType.DMA] * (5 * NSLOT),
    )
    acc, den = edge_k(h, a1, a2, src, dst)

    den3 = den.reshape(NC, N, 1)
    FR = 2000
    out = pl.pallas_call(
        _fin_body,
        grid=(N // FR,),
        in_specs=[
            pl.BlockSpec((NC, FR, D), lambda i: (0, i, 0)),
            pl.BlockSpec((NC, FR, 1), lambda i: (0, i, 0)),
        ],
        out_specs=pl.BlockSpec((FR, D), lambda i: (i, 0)),
        out_shape=jax.ShapeDtypeStruct((N, D), jnp.float32),
    )(acc, den3)
    return (out[:5000], out[5000:])
